# baseline (speedup 1.0000x reference)
import jax
import jax.numpy as jnp
from jax.experimental import pallas as pl
from jax.experimental.pallas import tpu as pltpu

_N = 128
_P = 128
_T = 10
_OFFSETS_STRIDE = 4.0
_KEY_ITEM_WEIGHT = 0.5
_IGNORE_BOUND = 1000.0
_BETA = 1.0 / _OFFSETS_STRIDE
_JC = 16


def _smooth_l1(pred, target):
    diff = jnp.abs(pred - target)
    return jnp.where(diff < _BETA, 0.5 * diff * diff / _BETA, diff - 0.5 * _BETA)


def _dm_kernel(
    pxf, pyf, oxf, oyf, kxf, kyf, mf,
    gxc, gyc, gxrc, gyrc, pxc, pyc, oxc, oyc,
    out_ref,
    runmin, seltx, selty, mn2, pselx, psely, oselx, osely,
):
    i = pl.program_id(0)

    @pl.when(i == 0)
    def _():
        runmin[...] = jnp.full((_P, _N), jnp.inf, jnp.float32)
        mn2[...] = jnp.full((_P, _N), jnp.inf, jnp.float32)

    gxr3 = gxrc[...][:, None, :]
    gyr3 = gyrc[...][:, None, :]
    bx3 = gxc[...][:, None, :] - gxr3
    by3 = gyc[...][:, None, :] - gyr3
    c3 = bx3 * bx3 + by3 * by3
    ncr = jnp.where(c3 > 1e-30, -float(_T) / c3, 0.0)

    px3 = pxf[...][None, :, :]
    py3 = pyf[...][None, :, :]
    dx = gxr3 - px3
    dy = gyr3 - py3
    a3 = dx * dx + dy * dy
    e3 = dx * bx3 + dy * by3
    e23 = e3 + e3

    xs = e3 * ncr
    sf = jnp.clip(jnp.floor(xs), 0.0, float(_T - 1))
    s2 = jnp.minimum(sf + 1.0, float(_T - 1))
    w1 = sf * (1.0 / _T)
    w2 = s2 * (1.0 / _T)
    d1 = (c3 * w1 + e23) * w1 + a3
    d2 = (c3 * w2 + e23) * w2 + a3
    take1 = d1 <= d2
    dmin = jnp.where(take1, d1, d2)
    w_at = jnp.where(take1, w1, w2)
    tx3 = bx3 * w_at + gxr3
    ty3 = by3 * w_at + gyr3

    rm = runmin[...]
    sx = seltx[...]
    sy = selty[...]
    for jj in range(_JC):
        upd = dmin[jj] < rm
        rm = jnp.where(upd, dmin[jj], rm)
        sx = jnp.where(upd, tx3[jj], sx)
        sy = jnp.where(upd, ty3[jj], sy)
    runmin[...] = rm
    seltx[...] = sx
    selty[...] = sy

    kx3 = kxf[...][None, :, :]
    ky3 = kyf[...][None, :, :]
    pxr = pxc[...]
    pyr = pyc[...]
    oxr = oxc[...]
    oyr = oyc[...]
    dx2 = pxr[:, None, :] - kx3
    dy2 = pyr[:, None, :] - ky3
    dd2 = dx2 * dx2 + dy2 * dy2

    m2v = mn2[...]
    qx = pselx[...]
    qy = psely[...]
    rx = oselx[...]
    ry = osely[...]
    for jj in range(_JC):
        upd = dd2[jj] < m2v
        m2v = jnp.where(upd, dd2[jj], m2v)
        qx = jnp.where(upd, pxr[jj][None, :], qx)
        qy = jnp.where(upd, pyr[jj][None, :], qy)
        rx = jnp.where(upd, oxr[jj][None, :], rx)
        ry = jnp.where(upd, oyr[jj][None, :], ry)
    mn2[...] = m2v
    pselx[...] = qx
    psely[...] = qy
    oselx[...] = rx
    osely[...] = ry

    @pl.when(i == pl.num_programs(0) - 1)
    def _():
        inv = 1.0 / _OFFSETS_STRIDE
        bound = _IGNORE_BOUND * _IGNORE_BOUND
        valid1 = rm <= bound
        sl1 = _smooth_l1(oxf[...], (sx - pxf[...]) * inv) + _smooth_l1(
            oyf[...], (sy - pyf[...]) * inv
        )
        s1 = jnp.sum(jnp.where(valid1, sl1, 0.0))
        c1 = jnp.sum(valid1.astype(jnp.float32))

        valid2 = m2v <= bound
        mk = jnp.logical_and(mf[...] > 0.0, valid2)
        sl2 = _smooth_l1(rx, (kxf[...] - qx) * inv) + _smooth_l1(
            ry, (kyf[...] - qy) * inv
        )
        s2s = jnp.sum(jnp.where(mk, sl2, 0.0))
        c2 = jnp.sum(mk.astype(jnp.float32))

        denom1 = jnp.maximum(c1 * 2.0, 1.0)
        denom2 = jnp.maximum(c2 * 2.0, 1.0)
        out_ref[0, 0] = (s1 / denom1) * (1.0 - _KEY_ITEM_WEIGHT) + (
            s2s / denom2
        ) * _KEY_ITEM_WEIGHT


def kernel(pred_contours, pred_offsets, gt_contours, gt_key_points, gt_key_points_mask):
    px = pred_contours[..., 0].T
    py = pred_contours[..., 1].T
    ox = pred_offsets[..., 0].T
    oy = pred_offsets[..., 1].T
    gx = gt_contours[..., 0].T
    gy = gt_contours[..., 1].T
    gxr = jnp.roll(gx, 1, axis=0)
    gyr = jnp.roll(gy, 1, axis=0)
    kx = gt_key_points[..., 0].T
    ky = gt_key_points[..., 1].T
    m = gt_key_points_mask.astype(jnp.float32).T

    full = pl.BlockSpec((_P, _N), lambda i: (0, 0))
    chunk = pl.BlockSpec((_JC, _N), lambda i: (i, 0))
    out = pl.pallas_call(
        _dm_kernel,
        grid=(_P // _JC,),
        in_specs=[full] * 7 + [chunk] * 8,
        out_specs=pl.BlockSpec(memory_space=pltpu.SMEM),
        out_shape=jax.ShapeDtypeStruct((1, 1), jnp.float32),
        scratch_shapes=[pltpu.VMEM((_P, _N), jnp.float32)] * 8,
    )(px, py, ox, oy, kx, ky, m, gx, gy, gxr, gyr, px, py, ox, oy)
    return out[0, 0]


# speedup vs baseline: 1.0732x; 1.0732x over previous
import jax
import jax.numpy as jnp
from jax.experimental import pallas as pl
from jax.experimental.pallas import tpu as pltpu

_N = 128
_P = 128
_T = 10
_OFFSETS_STRIDE = 4.0
_KEY_ITEM_WEIGHT = 0.5
_IGNORE_BOUND = 1000.0
_BETA = 1.0 / _OFFSETS_STRIDE
_JC = 8


def _smooth_l1(pred, target):
    diff = jnp.abs(pred - target)
    return jnp.where(diff < _BETA, 0.5 * diff * diff / _BETA, diff - 0.5 * _BETA)


def _dm_kernel(
    pxf, pyf, oxf, oyf, kxf, kyf, mf,
    gxc, gyc, gxrc, gyrc, pxc, pyc, oxc, oyc,
    out_ref,
    runmin, seltx, selty, mn2, pselx, psely, oselx, osely,
):
    i = pl.program_id(0)

    @pl.when(i == 0)
    def _():
        runmin[...] = jnp.full((_P, _N), jnp.inf, jnp.float32)
        mn2[...] = jnp.full((_P, _N), jnp.inf, jnp.float32)

    gxr3 = gxrc[...][:, None, :]
    gyr3 = gyrc[...][:, None, :]
    bx3 = gxc[...][:, None, :] - gxr3
    by3 = gyc[...][:, None, :] - gyr3
    c3 = bx3 * bx3 + by3 * by3
    ncr = jnp.where(c3 > 1e-30, -float(_T) / c3, 0.0)
    c100 = c3 * (1.0 / (_T * _T))
    c50 = c100 + c100
    bxT = bx3 * (1.0 / _T)
    byT = by3 * (1.0 / _T)

    px3 = pxf[...][None, :, :]
    py3 = pyf[...][None, :, :]
    dx = gxr3 - px3
    dy = gyr3 - py3
    a3 = dx * dx + dy * dy
    e3 = dx * bx3 + dy * by3

    xs = e3 * ncr
    sf = jnp.clip(jnp.floor(xs), 0.0, float(_T - 2))
    e5 = e3 * (2.0 / _T)
    d1 = (c100 * sf + e5) * sf + a3
    delta = c50 * sf + (c100 + e5)
    take1 = delta >= 0.0
    d2 = d1 + delta
    dmin = jnp.where(take1, d1, d2)
    s_at = jnp.where(take1, sf, sf + 1.0)
    tx3 = bxT * s_at + gxr3
    ty3 = byT * s_at + gyr3

    rm = runmin[...]
    sx = seltx[...]
    sy = selty[...]
    for jj in range(_JC):
        upd = dmin[jj] < rm
        rm = jnp.where(upd, dmin[jj], rm)
        sx = jnp.where(upd, tx3[jj], sx)
        sy = jnp.where(upd, ty3[jj], sy)
    runmin[...] = rm
    seltx[...] = sx
    selty[...] = sy

    kx3 = kxf[...][None, :, :]
    ky3 = kyf[...][None, :, :]
    pxr = pxc[...]
    pyr = pyc[...]
    oxr = oxc[...]
    oyr = oyc[...]
    dx2 = pxr[:, None, :] - kx3
    dy2 = pyr[:, None, :] - ky3
    dd2 = dx2 * dx2 + dy2 * dy2

    m2v = mn2[...]
    qx = pselx[...]
    qy = psely[...]
    rx = oselx[...]
    ry = osely[...]
    for jj in range(_JC):
        upd = dd2[jj] < m2v
        m2v = jnp.where(upd, dd2[jj], m2v)
        qx = jnp.where(upd, pxr[jj][None, :], qx)
        qy = jnp.where(upd, pyr[jj][None, :], qy)
        rx = jnp.where(upd, oxr[jj][None, :], rx)
        ry = jnp.where(upd, oyr[jj][None, :], ry)
    mn2[...] = m2v
    pselx[...] = qx
    psely[...] = qy
    oselx[...] = rx
    osely[...] = ry

    @pl.when(i == pl.num_programs(0) - 1)
    def _():
        inv = 1.0 / _OFFSETS_STRIDE
        bound = _IGNORE_BOUND * _IGNORE_BOUND
        valid1 = rm <= bound
        sl1 = _smooth_l1(oxf[...], (sx - pxf[...]) * inv) + _smooth_l1(
            oyf[...], (sy - pyf[...]) * inv
        )
        s1 = jnp.sum(jnp.where(valid1, sl1, 0.0))
        c1 = jnp.sum(valid1.astype(jnp.float32))

        valid2 = m2v <= bound
        mk = jnp.logical_and(mf[...] > 0.0, valid2)
        sl2 = _smooth_l1(rx, (kxf[...] - qx) * inv) + _smooth_l1(
            ry, (kyf[...] - qy) * inv
        )
        s2s = jnp.sum(jnp.where(mk, sl2, 0.0))
        c2 = jnp.sum(mk.astype(jnp.float32))

        denom1 = jnp.maximum(c1 * 2.0, 1.0)
        denom2 = jnp.maximum(c2 * 2.0, 1.0)
        out_ref[0, 0] = (s1 / denom1) * (1.0 - _KEY_ITEM_WEIGHT) + (
            s2s / denom2
        ) * _KEY_ITEM_WEIGHT


def kernel(pred_contours, pred_offsets, gt_contours, gt_key_points, gt_key_points_mask):
    px = pred_contours[..., 0].T
    py = pred_contours[..., 1].T
    ox = pred_offsets[..., 0].T
    oy = pred_offsets[..., 1].T
    gx = gt_contours[..., 0].T
    gy = gt_contours[..., 1].T
    gxr = jnp.roll(gx, 1, axis=0)
    gyr = jnp.roll(gy, 1, axis=0)
    kx = gt_key_points[..., 0].T
    ky = gt_key_points[..., 1].T
    m = gt_key_points_mask.astype(jnp.float32).T

    full = pl.BlockSpec((_P, _N), lambda i: (0, 0))
    chunk = pl.BlockSpec((_JC, _N), lambda i: (i, 0))
    out = pl.pallas_call(
        _dm_kernel,
        grid=(_P // _JC,),
        in_specs=[full] * 7 + [chunk] * 8,
        out_specs=pl.BlockSpec(memory_space=pltpu.SMEM),
        out_shape=jax.ShapeDtypeStruct((1, 1), jnp.float32),
        scratch_shapes=[pltpu.VMEM((_P, _N), jnp.float32)] * 8,
    )(px, py, ox, oy, kx, ky, m, gx, gy, gxr, gyr, px, py, ox, oy)
    return out[0, 0]
